# baseline (speedup 1.0000x reference)
import functools

import jax
import jax.numpy as jnp
from jax.experimental import pallas as pl
from jax.experimental.pallas import tpu as pltpu

_N = 50
_NP = 64
_F = 128
_D = 32
_BF = jnp.bfloat16


def _prep_body(users_ref, gamma_ref, beta_ref, Wue_ref, bue_ref, Wpe_ref,
               bpe_ref, Wg0_ref, Wu0_ref, bu0_ref, Wu1_ref, bu1_ref,
               Wp0_ref, bp0_ref, Wp1_ref, bp1_ref,
               as0_ref, ad0_ref, Wg1_ref, as1_ref, ad1_ref,
               mean_ref, ginv_ref, t_ref, vpost_ref, cu_ref, wp_ref, cp_ref,
               Wx0_ref, bx0_ref, ce0_ref, Wx1_ref, bx1_ref):
    u = users_ref[...]
    mean = jnp.mean(u, axis=0, keepdims=True)
    var = jnp.mean((u - mean) * (u - mean), axis=0, keepdims=True)
    ginv = gamma_ref[...] * jax.lax.rsqrt(var + 1e-5)
    mean_ref[...] = mean
    ginv_ref[...] = ginv

    dot = functools.partial(jnp.dot, preferred_element_type=jnp.float32)
    wu = dot(Wu0_ref[...], Wu1_ref[...])
    wu_top = wu[0:_D, :]
    t = dot(Wue_ref[...], wu_top)
    t_ref[...] = t
    vpost_ref[...] = wu[_D:2 * _D, :]
    cu_ref[...] = (dot(bu0_ref[...], Wu1_ref[...]) + bu1_ref[...]
                   + dot(beta_ref[...], t) + dot(bue_ref[...], wu_top))
    wp = dot(Wp0_ref[...], Wp1_ref[...])
    wp_ref[...] = wp
    cp_ref[...] = dot(bp0_ref[...], Wp1_ref[...]) + bp1_ref[...]
    Wg0e = dot(Wpe_ref[...], Wg0_ref[...])
    bg0 = dot(bpe_ref[...], Wg0_ref[...])
    zc = jnp.zeros((_F, 1), jnp.float32)
    Wx0_ref[...] = jnp.concatenate(
        [Wg0e, zc, dot(Wg0e, as0_ref[...]), dot(Wg0e, ad0_ref[...])],
        axis=1)
    bx0_ref[...] = jnp.concatenate(
        [bg0, jnp.ones((1, 1), jnp.float32)], axis=1)
    ce0_ref[...] = dot(bg0, as0_ref[...] + ad0_ref[...])
    Wg1 = Wg1_ref[...]
    zc1 = jnp.zeros((_D, 1), jnp.float32)
    Wx1_ref[...] = jnp.concatenate(
        [Wg1, zc1, dot(Wg1, as1_ref[...]), dot(Wg1, ad1_ref[...])],
        axis=1)
    bx1_ref[...] = jnp.concatenate(
        [jnp.zeros((1, _D), jnp.float32),
         jnp.ones((1, 1), jnp.float32)], axis=1)


def _dot2(a, b):
    return jax.lax.dot_general(
        a.astype(_BF), b.astype(_BF), (((1,), (0,)), ((), ())),
        preferred_element_type=jnp.float32)


def _bdot(a, b):
    return jax.lax.dot_general(
        a.astype(_BF), b.astype(_BF), (((2,), (1,)), ((0,), (0,))),
        preferred_element_type=jnp.float32)


def _attend(hx3, adj, ce, bx, ones_col):
    es = hx3[:, :, _D + 1:_D + 2]
    edc = hx3[:, :, _D + 2:_D + 3]
    ed = jax.lax.dot_general(
        ones_col[:, 0:1, :], edc, (((2,), (2,)), ((0,), (0,))),
        preferred_element_type=jnp.float32)
    e = es + (ed + ce)
    e = jnp.maximum(e, 0.2 * e)
    p = jnp.where(adj > 0, jnp.exp(e), 0.0)
    hwo = hx3[:, :, 0:_D + 1] + bx
    oext = _bdot(p, hwo)
    den = jnp.maximum(oext[:, :, _D:_D + 1], 1e-30)
    out = oext[:, :, 0:_D] / den
    return jnp.where(out > 0, out, jnp.exp(out) - 1.0)


def _main_body(posts_ref, adj_ref, users_ref, mean_ref, ginv_ref, t_ref,
               vpost_ref, cu_ref, wp_ref, cp_ref, Wx0_ref, bx0_ref,
               ce0_ref, Wx1_ref, bx1_ref, ul_ref, plab_ref, sc_p, sc_a):
    cb = posts_ref.shape[0]

    @pl.when(pl.program_id(0) == 0)
    def _init():
        sc_p[...] = jnp.zeros_like(sc_p)
        sc_a[...] = jnp.zeros_like(sc_a)

    sc_p[:, 0:_N, :] = posts_ref[...].astype(_BF)
    sc_a[:, 0:_N, 0:_N] = adj_ref[...].astype(_BF)

    posts = sc_p[...]
    adj = sc_a[...]
    ones_col = jnp.ones((cb, _NP, 1), jnp.float32)

    hx0 = _dot2(posts.reshape(cb * _NP, _F),
                Wx0_ref[...]).reshape(cb, _NP, _D + 3)
    h1 = _attend(hx0, adj, ce0_ref[...][None], bx0_ref[...][None], ones_col)

    hx1 = _dot2(h1.reshape(cb * _NP, _D),
                Wx1_ref[...]).reshape(cb, _NP, _D + 3)
    zero = jnp.zeros((1, 1, 1), jnp.float32)
    pe = _attend(hx1, adj, zero, bx1_ref[...][None], ones_col)

    pco = (_dot2(pe.reshape(cb * _NP, _D), wp_ref[...])
           .reshape(cb, _NP, 1) + cp_ref[...][None])
    plab_ref[...] = jax.nn.sigmoid(pco[:, 0:_N, :])

    mp = jnp.max(pe[:, 0:_N, :], axis=1)
    un = (users_ref[...] - mean_ref[...]) * ginv_ref[...]
    uco = (_dot2(un, t_ref[...]) + _dot2(mp, vpost_ref[...])
           + cu_ref[...])
    ul_ref[...] = jax.nn.sigmoid(uco)


def kernel(users, posts, post_adjs, up_masking, bn_gamma, bn_beta,
           W_user_enc, b_user_enc, W_post_enc, b_post_enc,
           W_gat0, a_src0, a_dst0, W_gat1, a_src1, a_dst1,
           W_pcls0, b_pcls0, W_pcls1, b_pcls1,
           W_ucls0, b_ucls0, W_ucls1, b_ucls1):
    B, F = users.shape
    N = posts.shape[1]
    D = W_gat0.shape[0]

    row = lambda v: v.reshape(1, -1)
    col = lambda v: v.reshape(-1, 1)
    f32 = jnp.float32

    prep_outs = (
        jax.ShapeDtypeStruct((1, F), f32),
        jax.ShapeDtypeStruct((1, F), f32),
        jax.ShapeDtypeStruct((F, 1), f32),
        jax.ShapeDtypeStruct((D, 1), f32),
        jax.ShapeDtypeStruct((1, 1), f32),
        jax.ShapeDtypeStruct((D, 1), f32),
        jax.ShapeDtypeStruct((1, 1), f32),
        jax.ShapeDtypeStruct((F, D + 3), f32),
        jax.ShapeDtypeStruct((1, D + 1), f32),
        jax.ShapeDtypeStruct((1, 1), f32),
        jax.ShapeDtypeStruct((D, D + 3), f32),
        jax.ShapeDtypeStruct((1, D + 1), f32),
    )
    (mean, ginv, t, vpost, cu, wp, cp, Wx0, bx0, ce0, Wx1,
     bx1) = pl.pallas_call(
        _prep_body, out_shape=prep_outs)(
            users, row(bn_gamma), row(bn_beta), W_user_enc, row(b_user_enc),
            W_post_enc, row(b_post_enc), W_gat0, W_ucls0, row(b_ucls0),
            W_ucls1, row(b_ucls1), W_pcls0, row(b_pcls0), W_pcls1,
            row(b_pcls1), col(a_src0), col(a_dst0), W_gat1,
            col(a_src1), col(a_dst1))

    CB = 128
    grid = (B // CB,)
    full = lambda shape: pl.BlockSpec(shape, lambda i: (0,) * len(shape))
    in_specs = [
        pl.BlockSpec((CB, N, F), lambda i: (i, 0, 0)),
        pl.BlockSpec((CB, N, N), lambda i: (i, 0, 0)),
        pl.BlockSpec((CB, F), lambda i: (i, 0)),
        full((1, F)), full((1, F)), full((F, 1)), full((D, 1)),
        full((1, 1)), full((D, 1)), full((1, 1)), full((F, D + 3)),
        full((1, D + 1)), full((1, 1)), full((D, D + 3)),
        full((1, D + 1)),
    ]
    out_specs = [
        pl.BlockSpec((CB, 1), lambda i: (i, 0)),
        pl.BlockSpec((CB, N, 1), lambda i: (i, 0, 0)),
    ]
    user_label, post_label = pl.pallas_call(
        _main_body,
        grid=grid,
        in_specs=in_specs,
        out_specs=out_specs,
        out_shape=(
            jax.ShapeDtypeStruct((B, 1), f32),
            jax.ShapeDtypeStruct((B, N, 1), f32),
        ),
        scratch_shapes=[
            pltpu.VMEM((CB, _NP, _F), _BF),
            pltpu.VMEM((CB, _NP, _NP), _BF),
        ],
        compiler_params=pltpu.CompilerParams(
            dimension_semantics=("arbitrary",)),
    )(posts, post_adjs, users, mean, ginv, t,
      vpost, cu, wp, cp, Wx0, bx0, ce0, Wx1, bx1)
    return (user_label, post_label)


# speedup vs baseline: 1.0703x; 1.0703x over previous
import functools

import jax
import jax.numpy as jnp
from jax.experimental import pallas as pl
from jax.experimental.pallas import tpu as pltpu

_N = 50
_NP = 64
_F = 128
_D = 32
_BF = jnp.bfloat16


def _prep_body(users_ref, gamma_ref, beta_ref, Wue_ref, bue_ref, Wpe_ref,
               bpe_ref, Wg0_ref, Wu0_ref, bu0_ref, Wu1_ref, bu1_ref,
               Wp0_ref, bp0_ref, Wp1_ref, bp1_ref,
               as0_ref, ad0_ref, Wg1_ref, as1_ref, ad1_ref,
               mean_ref, ginv_ref, t_ref, vpost_ref, cu_ref, wp_ref, cp_ref,
               Wx0_ref, bx0_ref, ce0_ref, Wx1_ref, bx1_ref):
    u = users_ref[...]
    mean = jnp.mean(u, axis=0, keepdims=True)
    var = jnp.mean((u - mean) * (u - mean), axis=0, keepdims=True)
    ginv = gamma_ref[...] * jax.lax.rsqrt(var + 1e-5)
    mean_ref[...] = mean
    ginv_ref[...] = ginv

    dot = functools.partial(jnp.dot, preferred_element_type=jnp.float32)
    wu = dot(Wu0_ref[...], Wu1_ref[...])
    wu_top = wu[0:_D, :]
    t = dot(Wue_ref[...], wu_top)
    t_ref[...] = t
    vpost_ref[...] = wu[_D:2 * _D, :]
    cu_ref[...] = (dot(bu0_ref[...], Wu1_ref[...]) + bu1_ref[...]
                   + dot(beta_ref[...], t) + dot(bue_ref[...], wu_top))
    wp = dot(Wp0_ref[...], Wp1_ref[...])
    wp_ref[...] = wp
    cp_ref[...] = dot(bp0_ref[...], Wp1_ref[...]) + bp1_ref[...]
    Wg0e = dot(Wpe_ref[...], Wg0_ref[...])
    bg0 = dot(bpe_ref[...], Wg0_ref[...])
    zc = jnp.zeros((_F, 1), jnp.float32)
    Wx0_ref[...] = jnp.concatenate(
        [Wg0e, zc, dot(Wg0e, as0_ref[...]), dot(Wg0e, ad0_ref[...])],
        axis=1)
    bx0_ref[...] = jnp.concatenate(
        [bg0, jnp.ones((1, 1), jnp.float32)], axis=1)
    ce0_ref[...] = dot(bg0, as0_ref[...] + ad0_ref[...])
    Wg1 = Wg1_ref[...]
    zc1 = jnp.zeros((_D, 1), jnp.float32)
    Wx1_ref[...] = jnp.concatenate(
        [Wg1, zc1, dot(Wg1, as1_ref[...]), dot(Wg1, ad1_ref[...])],
        axis=1)
    bx1_ref[...] = jnp.concatenate(
        [jnp.zeros((1, _D), jnp.float32),
         jnp.ones((1, 1), jnp.float32)], axis=1)


def _dot2(a, b):
    return jax.lax.dot_general(
        a.astype(_BF), b.astype(_BF), (((1,), (0,)), ((), ())),
        preferred_element_type=jnp.float32)


def _bdot(a, b):
    return jax.lax.dot_general(
        a.astype(_BF), b.astype(_BF), (((2,), (1,)), ((0,), (0,))),
        preferred_element_type=jnp.float32)


def _attend(hx3, adj, ce, bx, ones_col):
    es = hx3[:, :, _D + 1:_D + 2]
    edc = hx3[:, :, _D + 2:_D + 3]
    ed = jax.lax.dot_general(
        ones_col[:, 0:1, :], edc, (((2,), (2,)), ((0,), (0,))),
        preferred_element_type=jnp.float32).astype(_BF)
    e = es + (ed + ce.astype(_BF))
    e = jnp.maximum(e, _BF(0.2) * e)
    p = jnp.where(adj > 0, jnp.exp(e), _BF(0.0))
    hwo = hx3[:, :, 0:_D + 1] + bx.astype(_BF)
    oext = _bdot(p, hwo)
    den = jnp.maximum(oext[:, :, _D:_D + 1], 1e-30)
    out = oext[:, :, 0:_D] / den
    return jnp.where(out > 0, out, jnp.exp(out) - 1.0)


def _main_body(posts_ref, adj_ref, users_ref, mean_ref, ginv_ref, t_ref,
               vpost_ref, cu_ref, wp_ref, cp_ref, Wx0_ref, bx0_ref,
               ce0_ref, Wx1_ref, bx1_ref, ul_ref, plab_ref, sc_p, sc_a):
    cb = posts_ref.shape[0]

    @pl.when(pl.program_id(0) == 0)
    def _init():
        sc_p[...] = jnp.zeros_like(sc_p)
        sc_a[...] = jnp.zeros_like(sc_a)

    sc_p[:, 0:_N, :] = posts_ref[...].astype(_BF)
    sc_a[:, 0:_N, 0:_N] = adj_ref[...].astype(_BF)

    posts = sc_p[...]
    adj = sc_a[...]
    ones_col = jnp.ones((cb, _NP, 1), _BF)

    hx0 = _dot2(posts.reshape(cb * _NP, _F),
                Wx0_ref[...]).astype(_BF).reshape(cb, _NP, _D + 3)
    h1 = _attend(hx0, adj, ce0_ref[...][None], bx0_ref[...][None], ones_col)

    hx1 = _dot2(h1.reshape(cb * _NP, _D),
                Wx1_ref[...]).astype(_BF).reshape(cb, _NP, _D + 3)
    zero = jnp.zeros((1, 1, 1), jnp.float32)
    pe = _attend(hx1, adj, zero, bx1_ref[...][None], ones_col)

    pco = (_dot2(pe.reshape(cb * _NP, _D), wp_ref[...])
           .reshape(cb, _NP, 1) + cp_ref[...][None])
    plab_ref[...] = jax.nn.sigmoid(pco[:, 0:_N, :])

    mp = jnp.max(pe[:, 0:_N, :], axis=1)
    un = (users_ref[...] - mean_ref[...]) * ginv_ref[...]
    uco = (_dot2(un, t_ref[...]) + _dot2(mp, vpost_ref[...])
           + cu_ref[...])
    ul_ref[...] = jax.nn.sigmoid(uco)


def kernel(users, posts, post_adjs, up_masking, bn_gamma, bn_beta,
           W_user_enc, b_user_enc, W_post_enc, b_post_enc,
           W_gat0, a_src0, a_dst0, W_gat1, a_src1, a_dst1,
           W_pcls0, b_pcls0, W_pcls1, b_pcls1,
           W_ucls0, b_ucls0, W_ucls1, b_ucls1):
    B, F = users.shape
    N = posts.shape[1]
    D = W_gat0.shape[0]

    row = lambda v: v.reshape(1, -1)
    col = lambda v: v.reshape(-1, 1)
    f32 = jnp.float32

    prep_outs = (
        jax.ShapeDtypeStruct((1, F), f32),
        jax.ShapeDtypeStruct((1, F), f32),
        jax.ShapeDtypeStruct((F, 1), f32),
        jax.ShapeDtypeStruct((D, 1), f32),
        jax.ShapeDtypeStruct((1, 1), f32),
        jax.ShapeDtypeStruct((D, 1), f32),
        jax.ShapeDtypeStruct((1, 1), f32),
        jax.ShapeDtypeStruct((F, D + 3), f32),
        jax.ShapeDtypeStruct((1, D + 1), f32),
        jax.ShapeDtypeStruct((1, 1), f32),
        jax.ShapeDtypeStruct((D, D + 3), f32),
        jax.ShapeDtypeStruct((1, D + 1), f32),
    )
    (mean, ginv, t, vpost, cu, wp, cp, Wx0, bx0, ce0, Wx1,
     bx1) = pl.pallas_call(
        _prep_body, out_shape=prep_outs)(
            users, row(bn_gamma), row(bn_beta), W_user_enc, row(b_user_enc),
            W_post_enc, row(b_post_enc), W_gat0, W_ucls0, row(b_ucls0),
            W_ucls1, row(b_ucls1), W_pcls0, row(b_pcls0), W_pcls1,
            row(b_pcls1), col(a_src0), col(a_dst0), W_gat1,
            col(a_src1), col(a_dst1))

    CB = 128
    grid = (B // CB,)
    full = lambda shape: pl.BlockSpec(shape, lambda i: (0,) * len(shape))
    in_specs = [
        pl.BlockSpec((CB, N, F), lambda i: (i, 0, 0)),
        pl.BlockSpec((CB, N, N), lambda i: (i, 0, 0)),
        pl.BlockSpec((CB, F), lambda i: (i, 0)),
        full((1, F)), full((1, F)), full((F, 1)), full((D, 1)),
        full((1, 1)), full((D, 1)), full((1, 1)), full((F, D + 3)),
        full((1, D + 1)), full((1, 1)), full((D, D + 3)),
        full((1, D + 1)),
    ]
    out_specs = [
        pl.BlockSpec((CB, 1), lambda i: (i, 0)),
        pl.BlockSpec((CB, N, 1), lambda i: (i, 0, 0)),
    ]
    user_label, post_label = pl.pallas_call(
        _main_body,
        grid=grid,
        in_specs=in_specs,
        out_specs=out_specs,
        out_shape=(
            jax.ShapeDtypeStruct((B, 1), f32),
            jax.ShapeDtypeStruct((B, N, 1), f32),
        ),
        scratch_shapes=[
            pltpu.VMEM((CB, _NP, _F), _BF),
            pltpu.VMEM((CB, _NP, _NP), _BF),
        ],
        compiler_params=pltpu.CompilerParams(
            dimension_semantics=("arbitrary",)),
    )(posts, post_adjs, users, mean, ginv, t,
      vpost, cu, wp, cp, Wx0, bx0, ce0, Wx1, bx1)
    return (user_label, post_label)
